# baseline (speedup 1.0000x reference)
import functools

import jax
import jax.numpy as jnp
from jax import lax
from jax.experimental import pallas as pl
from jax.experimental.pallas import tpu as pltpu
from jax.experimental.pallas import tpu_sc as plsc

SEQ = 200
D = 64
NUM_CORES = 2
NUM_SUBCORES = 16
NUM_WORKERS = NUM_CORES * NUM_SUBCORES
LANES = 16
G0, G1 = 128, 72


def kernel(inputs, token_table, position_table):
    batch, seq = inputs.shape
    vocab, d = token_table.shape
    total = batch * seq
    rows_per_w = total // NUM_WORKERS
    seq_per_w = rows_per_w // seq

    idx_flat = inputs.reshape(total).astype(jnp.int32)

    mesh = plsc.VectorSubcoreMesh(core_axis_name="c", subcore_axis_name="s")

    @functools.partial(
        pl.kernel,
        out_type=jax.ShapeDtypeStruct((total, d), jnp.float32),
        mesh=mesh,
        scratch_types=[
            pltpu.VMEM((rows_per_w,), jnp.int32),
            pltpu.VMEM((SEQ, D), jnp.float32),
            pltpu.VMEM((SEQ, D), jnp.float32),
            pltpu.VMEM((SEQ, D), jnp.float32),
            pltpu.SemaphoreType.DMA,
            pltpu.SemaphoreType.DMA,
            pltpu.SemaphoreType.DMA,
            pltpu.SemaphoreType.DMA,
        ],
        compiler_params=pltpu.CompilerParams(use_tc_tiling_on_sc=False),
    )
    def sc_embed(idx_hbm, tab_hbm, pos_hbm, out_hbm, idx_v, pos_v, rows0,
                 rows1, sem_g0, sem_g1, sem_o0, sem_o1):
        wid = lax.axis_index("s") * NUM_CORES + lax.axis_index("c")
        base = wid * rows_per_w

        def issue_gather(s, rows_b, sem):
            o = s * SEQ
            pltpu.async_copy(
                tab_hbm.at[idx_v.at[pl.ds(o, G0)]], rows_b.at[pl.ds(0, G0)], sem
            )
            pltpu.async_copy(
                tab_hbm.at[idx_v.at[pl.ds(o + G0, G1)]],
                rows_b.at[pl.ds(G0, G1)], sem,
            )

        def wait_gather(s, rows_b, sem):
            o = s * SEQ
            pltpu.make_async_copy(
                tab_hbm.at[idx_v.at[pl.ds(o, G0)]], rows_b.at[pl.ds(0, G0)], sem
            ).wait()
            pltpu.make_async_copy(
                tab_hbm.at[idx_v.at[pl.ds(o + G0, G1)]],
                rows_b.at[pl.ds(G0, G1)], sem,
            ).wait()

        def issue_out(s, rows_b, sem):
            pltpu.async_copy(rows_b, out_hbm.at[pl.ds(base + s * SEQ, SEQ)], sem)

        def wait_out(s, rows_b, sem):
            pltpu.make_async_copy(
                rows_b, out_hbm.at[pl.ds(base + s * SEQ, SEQ)], sem
            ).wait()

        def add_pos(rows_b):
            @pl.loop(0, SEQ)
            def _(r):
                for j in range(D // LANES):
                    sl = (pl.ds(r, 1), pl.ds(j * LANES, LANES))
                    plsc.addupdate(rows_b.at[sl], pos_v.at[sl][...])

        pltpu.sync_copy(idx_hbm.at[pl.ds(base, rows_per_w)], idx_v)
        pltpu.sync_copy(pos_hbm, pos_v)

        issue_gather(0, rows0, sem_g0)
        issue_gather(1, rows1, sem_g1)
        wait_gather(0, rows0, sem_g0)
        add_pos(rows0)
        issue_out(0, rows0, sem_o0)

        @pl.loop(0, (seq_per_w - 2) // 2)
        def _(i):
            s1 = 2 * i + 1
            wait_out(s1 - 1, rows0, sem_o0)
            issue_gather(s1 + 1, rows0, sem_g0)
            wait_gather(s1, rows1, sem_g1)
            add_pos(rows1)
            issue_out(s1, rows1, sem_o1)

            s2 = 2 * i + 2
            wait_out(s2 - 1, rows1, sem_o1)
            issue_gather(s2 + 1, rows1, sem_g1)
            wait_gather(s2, rows0, sem_g0)
            add_pos(rows0)
            issue_out(s2, rows0, sem_o0)

        last = seq_per_w - 1
        wait_out(last - 1, rows0, sem_o0)
        wait_gather(last, rows1, sem_g1)
        add_pos(rows1)
        issue_out(last, rows1, sem_o1)
        wait_out(last, rows1, sem_o1)

    out = sc_embed(idx_flat, token_table, position_table)
    return out.reshape(batch, seq, d)


# speedup vs baseline: 1.3873x; 1.3873x over previous
import functools

import jax
import jax.numpy as jnp
from jax import lax
from jax.experimental import pallas as pl
from jax.experimental.pallas import tpu as pltpu
from jax.experimental.pallas import tpu_sc as plsc

SEQ = 200
D = 64
DP = 128
NUM_CORES = 2
NUM_SUBCORES = 16
NUM_WORKERS = NUM_CORES * NUM_SUBCORES
LANES = 16
G0, G1 = 128, 72


def kernel(inputs, token_table, position_table):
    batch, seq = inputs.shape
    vocab, d = token_table.shape
    total = batch * seq
    rows_per_w = total // NUM_WORKERS
    seq_per_w = rows_per_w // seq

    idx_flat = inputs.reshape(total).astype(jnp.int32)
    tab_p = jnp.pad(token_table, ((0, 0), (0, DP - d)))
    pos_p = jnp.pad(position_table, ((0, 0), (0, DP - d)))

    mesh = plsc.VectorSubcoreMesh(core_axis_name="c", subcore_axis_name="s")

    @functools.partial(
        pl.kernel,
        out_type=jax.ShapeDtypeStruct((batch, seq, DP), jnp.float32),
        mesh=mesh,
        scratch_types=[
            pltpu.VMEM((rows_per_w,), jnp.int32),
            pltpu.VMEM((SEQ, DP), jnp.float32),
            pltpu.VMEM((SEQ, DP), jnp.float32),
            pltpu.VMEM((SEQ, DP), jnp.float32),
            pltpu.SemaphoreType.DMA,
            pltpu.SemaphoreType.DMA,
            pltpu.SemaphoreType.DMA,
            pltpu.SemaphoreType.DMA,
        ],
        compiler_params=pltpu.CompilerParams(use_tc_tiling_on_sc=True),
    )
    def sc_embed(idx_hbm, tab_hbm, pos_hbm, out_hbm, idx_v, pos_v, rows0,
                 rows1, sem_g0, sem_g1, sem_o0, sem_o1):
        wid = lax.axis_index("s") * NUM_CORES + lax.axis_index("c")
        base = wid * rows_per_w

        def issue_gather(s, rows_b, sem):
            o = s * SEQ
            pltpu.async_copy(
                tab_hbm.at[idx_v.at[pl.ds(o, G0)]], rows_b.at[pl.ds(0, G0)], sem
            )
            pltpu.async_copy(
                tab_hbm.at[idx_v.at[pl.ds(o + G0, G1)]],
                rows_b.at[pl.ds(G0, G1)], sem,
            )

        def wait_gather(s, rows_b, sem):
            o = s * SEQ
            pltpu.make_async_copy(
                tab_hbm.at[idx_v.at[pl.ds(o, G0)]], rows_b.at[pl.ds(0, G0)], sem
            ).wait()
            pltpu.make_async_copy(
                tab_hbm.at[idx_v.at[pl.ds(o + G0, G1)]],
                rows_b.at[pl.ds(G0, G1)], sem,
            ).wait()

        def issue_out(s, rows_b, sem):
            pltpu.async_copy(rows_b, out_hbm.at[wid * seq_per_w + s], sem)

        def wait_out(s, rows_b, sem):
            pltpu.make_async_copy(
                rows_b, out_hbm.at[wid * seq_per_w + s], sem
            ).wait()

        def add_pos(rows_b):
            @pl.loop(0, SEQ)
            def _(r):
                for j in range(D // LANES):
                    sl = (pl.ds(r, 1), pl.ds(j * LANES, LANES))
                    plsc.addupdate(rows_b.at[sl], pos_v.at[sl][...])

        pltpu.sync_copy(idx_hbm.at[pl.ds(base, rows_per_w)], idx_v)
        pltpu.sync_copy(pos_hbm, pos_v)

        issue_gather(0, rows0, sem_g0)
        issue_gather(1, rows1, sem_g1)
        wait_gather(0, rows0, sem_g0)
        add_pos(rows0)
        issue_out(0, rows0, sem_o0)

        @pl.loop(0, (seq_per_w - 2) // 2)
        def _(i):
            s1 = 2 * i + 1
            wait_out(s1 - 1, rows0, sem_o0)
            issue_gather(s1 + 1, rows0, sem_g0)
            wait_gather(s1, rows1, sem_g1)
            add_pos(rows1)
            issue_out(s1, rows1, sem_o1)

            s2 = 2 * i + 2
            wait_out(s2 - 1, rows1, sem_o1)
            issue_gather(s2 + 1, rows1, sem_g1)
            wait_gather(s2, rows0, sem_g0)
            add_pos(rows0)
            issue_out(s2, rows0, sem_o0)

        last = seq_per_w - 1
        wait_out(last - 1, rows0, sem_o0)
        wait_gather(last, rows1, sem_g1)
        add_pos(rows1)
        issue_out(last, rows1, sem_o1)
        wait_out(last, rows1, sem_o1)

    out = sc_embed(idx_flat, tab_p, pos_p)
    return out[:, :, :D]


# speedup vs baseline: 1.3893x; 1.0014x over previous
import functools

import jax
import jax.numpy as jnp
from jax import lax
from jax.experimental import pallas as pl
from jax.experimental.pallas import tpu as pltpu
from jax.experimental.pallas import tpu_sc as plsc

SEQ = 200
D = 64
DP = 128
NUM_CORES = 2
NUM_SUBCORES = 16
NUM_WORKERS = NUM_CORES * NUM_SUBCORES
LANES = 16
NBUF = 3
G0, G1 = 128, 72


def kernel(inputs, token_table, position_table):
    batch, seq = inputs.shape
    vocab, d = token_table.shape
    total = batch * seq
    rows_per_w = total // NUM_WORKERS
    seq_per_w = rows_per_w // seq

    idx_flat = inputs.reshape(total).astype(jnp.int32)
    tab_p = jnp.pad(token_table, ((0, 0), (0, DP - d)))
    pos_p = jnp.pad(position_table, ((0, 0), (0, DP - d)))

    mesh = plsc.VectorSubcoreMesh(core_axis_name="c", subcore_axis_name="s")

    @functools.partial(
        pl.kernel,
        out_type=jax.ShapeDtypeStruct((batch, seq, DP), jnp.float32),
        mesh=mesh,
        scratch_types=[
            pltpu.VMEM((rows_per_w,), jnp.int32),
            pltpu.VMEM((SEQ, DP), jnp.float32),
            pltpu.VMEM((SEQ, DP), jnp.float32),
            pltpu.VMEM((SEQ, DP), jnp.float32),
            pltpu.VMEM((SEQ, DP), jnp.float32),
            pltpu.SemaphoreType.DMA,
            pltpu.SemaphoreType.DMA,
            pltpu.SemaphoreType.DMA,
            pltpu.SemaphoreType.DMA,
            pltpu.SemaphoreType.DMA,
            pltpu.SemaphoreType.DMA,
        ],
        compiler_params=pltpu.CompilerParams(use_tc_tiling_on_sc=True),
    )
    def sc_embed(idx_hbm, tab_hbm, pos_hbm, out_hbm, idx_v, pos_v, rows0,
                 rows1, rows2, g0, g1, g2, o0, o1, o2):
        wid = lax.axis_index("s") * NUM_CORES + lax.axis_index("c")
        base = wid * rows_per_w
        rows = (rows0, rows1, rows2)
        sem_g = (g0, g1, g2)
        sem_o = (o0, o1, o2)

        def issue_gather(s, b):
            o = s * SEQ
            pltpu.async_copy(
                tab_hbm.at[idx_v.at[pl.ds(o, G0)]], rows[b].at[pl.ds(0, G0)],
                sem_g[b],
            )
            pltpu.async_copy(
                tab_hbm.at[idx_v.at[pl.ds(o + G0, G1)]],
                rows[b].at[pl.ds(G0, G1)], sem_g[b],
            )

        def wait_gather(s, b):
            o = s * SEQ
            pltpu.make_async_copy(
                tab_hbm.at[idx_v.at[pl.ds(o, G0)]], rows[b].at[pl.ds(0, G0)],
                sem_g[b],
            ).wait()
            pltpu.make_async_copy(
                tab_hbm.at[idx_v.at[pl.ds(o + G0, G1)]],
                rows[b].at[pl.ds(G0, G1)], sem_g[b],
            ).wait()

        def issue_out(s, b):
            pltpu.async_copy(rows[b], out_hbm.at[wid * seq_per_w + s],
                             sem_o[b])

        def wait_out(s, b):
            pltpu.make_async_copy(
                rows[b], out_hbm.at[wid * seq_per_w + s], sem_o[b]
            ).wait()

        def add_pos(b):
            @pl.loop(0, SEQ)
            def _(r):
                for j in range(D // LANES):
                    sl = (pl.ds(r, 1), pl.ds(j * LANES, LANES))
                    plsc.addupdate(rows[b].at[sl], pos_v.at[sl][...])

        pltpu.sync_copy(idx_hbm.at[pl.ds(base, rows_per_w)], idx_v)
        pltpu.sync_copy(pos_hbm, pos_v)

        issue_gather(0, 0)
        issue_gather(1, 1)
        wait_gather(0, 0)
        add_pos(0)
        issue_out(0, 0)
        issue_gather(2, 2)
        wait_gather(1, 1)
        add_pos(1)
        issue_out(1, 1)
        wait_out(0, 0)
        issue_gather(3, 0)

        @pl.loop(0, (seq_per_w - 2) // NBUF)
        def _(i):
            for k in range(NBUF):
                s = NBUF * i + 2 + k
                b = (2 + k) % NBUF
                z = (1 + k) % NBUF
                wait_gather(s, b)
                add_pos(b)
                issue_out(s, b)
                wait_out(s - 1, z)

                @pl.when(s + 2 < seq_per_w)
                def _():
                    issue_gather(s + 2, z)

        wait_out(seq_per_w - 1, (seq_per_w - 1) % NBUF)

    out = sc_embed(idx_flat, tab_p, pos_p)
    return out[:, :, :D]
